# baseline (speedup 1.0000x reference)
import math

import jax
import jax.numpy as jnp
from jax import lax
from jax.experimental import pallas as pl
from jax.experimental.pallas import tpu as pltpu
from jax.experimental.pallas import tpu_sc as plsc

D = 64
VOCAB = 1000000
B_TOTAL = 1024 * 200
NC, NS, L = 2, 16, 16
NW = NC * NS
ROWS_W = B_TOTAL // NW
GROUP = 128
NGROUP = ROWS_W // GROUP
NBUF = 2
SEQ = 200
NSEG = 3
BLK_PER_W = 244
SCALE = 8.0


def _sinusoidal_pe(seq_len, d_model):
    pos = jnp.arange(seq_len, dtype=jnp.float32)[:, None]
    div = jnp.exp(
        jnp.arange(0, d_model, 2, dtype=jnp.float32)
        * (-math.log(10000.0) / d_model)
    )
    pe = jnp.zeros((seq_len, d_model), dtype=jnp.float32)
    pe = pe.at[:, 0::2].set(jnp.sin(pos * div))
    pe = pe.at[:, 1::2].set(jnp.cos(pos * div))
    return pe


def _diag_transpose(inb, outb, iota, njb):
    rot = [(iota + k) % L for k in range(L)]

    def sub_body(m, carry):
        c = m // njb
        jb = (m % njb) * L
        jv = jb + iota
        for k in range(L):
            dv = c * L + rot[k]
            v = plsc.load_gather(inb, [dv, jv])
            plsc.store_scatter(outb, [jv, dv], v)
        return carry
    lax.fori_loop(0, (D // L) * njb, sub_body, 0, unroll=4)


def _transpose_body(ttT_hbm, tok2_hbm, inb0, inb1, outb0, outb1,
                    inb_t, outb_t, gs0, gs1, ss0, ss1):
    inb = (inb0, inb1)
    outb = (outb0, outb1)
    gs = (gs0, gs1)
    ss = (ss0, ss1)
    wid = lax.axis_index("s") * NC + lax.axis_index("c")
    nb = 244 + jnp.where(wid < 4, 1, 0)
    lo = wid * BLK_PER_W + lax.min(wid, 4)
    iota = lax.iota(jnp.int32, L)

    def col0_of(i):
        return pl.multiple_of((lo + i) * 128, 128)

    def gather_start(i, b):
        pltpu.make_async_copy(
            ttT_hbm.at[:, pl.ds(col0_of(i), 128)], inb[b], gs[b]).start()

    def gather_wait(i, b):
        pltpu.make_async_copy(
            ttT_hbm.at[:, pl.ds(col0_of(i), 128)], inb[b], gs[b]).wait()

    def scatter_start(i, b):
        pltpu.make_async_copy(
            outb[b], tok2_hbm.at[pl.ds(col0_of(i), 128)], ss[b]).start()

    def scatter_wait(i, b):
        pltpu.make_async_copy(
            outb[b], tok2_hbm.at[pl.ds(col0_of(i), 128)], ss[b]).wait()

    for b in range(NBUF):
        gather_start(b, b)

    def blk_body(i2, carry):
        for b in range(NBUF):
            i = i2 * NBUF + b
            @pl.when(i < nb)
            def _():
                gather_wait(i, b)
                @pl.when(i2 > 0)
                def _():
                    scatter_wait(i - NBUF, b)
                _diag_transpose(inb[b], outb[b], iota, 8)
                @pl.when(i + NBUF < nb)
                def _():
                    gather_start(i + NBUF, b)
                scatter_start(i, b)
        return carry
    lax.fori_loop(0, (245 + NBUF - 1) // NBUF, blk_body, 0)

    for b in range(NBUF):
        scatter_wait(nb - NBUF + b, b)

    @pl.when(wid == NW - 1)
    def _():
        pltpu.sync_copy(ttT_hbm.at[:, pl.ds(VOCAB - D, D)], inb_t)
        _diag_transpose(inb_t, outb_t, iota, 4)
        pltpu.sync_copy(outb_t, tok2_hbm.at[pl.ds(VOCAB - D, D)])


def _gather_body(x_hbm, seg_hbm, tok2_hbm, pe_hbm, st_hbm, out_hbm,
                 idx_v, seg_v, comb_v, pe_v, st_v,
                 tb0, tb1, ob0, ob1, gs0, gs1, ss0, ss1):
    tokb = (tb0, tb1)
    outb = (ob0, ob1)
    gs = (gs0, gs1)
    ss = (ss0, ss1)
    wid = lax.axis_index("s") * NC + lax.axis_index("c")
    base = wid * ROWS_W

    pltpu.sync_copy(x_hbm.at[pl.ds(base, ROWS_W)], idx_v)
    pltpu.sync_copy(seg_hbm.at[pl.ds(base, ROWS_W)], seg_v)
    pltpu.sync_copy(pe_hbm, pe_v)
    pltpu.sync_copy(st_hbm, st_v)

    iota = lax.iota(jnp.int32, L)
    iota_c = [c * L + iota for c in range(D // L)]

    def comb_body(j, carry):
        pev = pe_v[pl.ds(j * L, L)]
        coff = (j % (D // L)) * L
        for s in range(NSEG):
            stv = st_v[pl.ds(s * D + coff, L)]
            comb_v[pl.ds(s * SEQ * D + j * L, L)] = pev + stv
        return carry
    lax.fori_loop(0, SEQ * D // L, comb_body, 0)

    def gathers_start(g, b):
        pltpu.make_async_copy(
            tok2_hbm.at[idx_v.at[pl.ds(g * GROUP, GROUP)]],
            tokb[b], gs[b]).start()

    def gathers_wait(g, b):
        pltpu.make_async_copy(
            tok2_hbm.at[idx_v.at[pl.ds(g * GROUP, GROUP)]],
            tokb[b], gs[b]).wait()

    def orow0_of(g):
        return pl.multiple_of(wid * (ROWS_W // 2) + g * (GROUP // 2),
                              GROUP // 2)

    def scatter_start(g, b):
        pltpu.make_async_copy(
            outb[b], out_hbm.at[pl.ds(orow0_of(g), GROUP // 2)],
            ss[b]).start()

    def scatter_wait(g, b):
        pltpu.make_async_copy(
            outb[b], out_hbm.at[pl.ds(orow0_of(g), GROUP // 2)],
            ss[b]).wait()

    for b in range(NBUF):
        gathers_start(b, b)

    def outer(og, carry):
        for b in range(NBUF):
            g = og * NBUF + b
            gathers_wait(g, b)
            @pl.when(og > 0)
            def _():
                scatter_wait(g - NBUF, b)

            gbase = g * GROUP

            def row_body(t, rcarry):
                r = gbase + t
                vseg16 = seg_v[pl.ds(r - r % L, L)]
                vseg = vseg16[jnp.full((L,), r % L, jnp.int32)]
                cbase = vseg * (SEQ * D) + (r % SEQ) * D
                orow = t // 2
                ocol = (t % 2) * D
                for c in range(D // L):
                    tok = tokb[b][t, pl.ds(c * L, L)]
                    cv = plsc.load_gather(comb_v, [cbase + iota_c[c]])
                    outb[b][orow, pl.ds(ocol + c * L, L)] = tok * SCALE + cv
                return rcarry
            lax.fori_loop(0, GROUP, row_body, 0, unroll=4)

            @pl.when(g + NBUF < NGROUP)
            def _():
                gathers_start(g + NBUF, b)
            scatter_start(g, b)
        return carry
    lax.fori_loop(0, NGROUP // NBUF, outer, 0)

    for b in range(NBUF):
        scatter_wait(NGROUP - NBUF + b, b)


def _sc_embed(x_flat, seg_flat, ttT, pe_flat, st_flat):
    mesh = plsc.VectorSubcoreMesh(core_axis_name="c", subcore_axis_name="s")
    params = pltpu.CompilerParams(
        use_tc_tiling_on_sc=True, needs_layout_passes=False)

    k1 = pl.kernel(
        _transpose_body,
        out_type=jax.ShapeDtypeStruct((VOCAB, 128), jnp.float32),
        mesh=mesh,
        scratch_types=[
            pltpu.VMEM((D, 128), jnp.float32),
            pltpu.VMEM((D, 128), jnp.float32),
            pltpu.VMEM((128, 128), jnp.float32),
            pltpu.VMEM((128, 128), jnp.float32),
            pltpu.VMEM((D, D), jnp.float32),
            pltpu.VMEM((D, 128), jnp.float32),
            pltpu.SemaphoreType.DMA,
            pltpu.SemaphoreType.DMA,
            pltpu.SemaphoreType.DMA,
            pltpu.SemaphoreType.DMA,
        ],
        compiler_params=params,
    )
    tok2 = k1(ttT)

    k2 = pl.kernel(
        _gather_body,
        out_type=jax.ShapeDtypeStruct((B_TOTAL // 2, 128), jnp.float32),
        mesh=mesh,
        scratch_types=[
            pltpu.VMEM((ROWS_W,), jnp.int32),
            pltpu.VMEM((ROWS_W,), jnp.int32),
            pltpu.VMEM((NSEG * SEQ * D,), jnp.float32),
            pltpu.VMEM((SEQ * D,), jnp.float32),
            pltpu.VMEM((NSEG * D,), jnp.float32),
            pltpu.VMEM((GROUP, 128), jnp.float32),
            pltpu.VMEM((GROUP, 128), jnp.float32),
            pltpu.VMEM((GROUP // 2, 128), jnp.float32),
            pltpu.VMEM((GROUP // 2, 128), jnp.float32),
            pltpu.SemaphoreType.DMA,
            pltpu.SemaphoreType.DMA,
            pltpu.SemaphoreType.DMA,
            pltpu.SemaphoreType.DMA,
        ],
        compiler_params=params,
    )
    return k2(x_flat, seg_flat, tok2, pe_flat, st_flat)


def kernel(x, segment_label, token_table, segment_table):
    batch, seq = x.shape
    x_flat = x.reshape(-1).astype(jnp.int32)
    seg_flat = segment_label.reshape(-1).astype(jnp.int32)
    pe_flat = _sinusoidal_pe(seq, D).reshape(-1)
    st_flat = segment_table.reshape(-1)
    out2 = _sc_embed(x_flat, seg_flat, token_table.T, pe_flat, st_flat)
    return out2.reshape(batch, seq, D)


# speedup vs baseline: 1.0375x; 1.0375x over previous
import math

import jax
import jax.numpy as jnp
from jax import lax
from jax.experimental import pallas as pl
from jax.experimental.pallas import tpu as pltpu
from jax.experimental.pallas import tpu_sc as plsc

D = 64
B_TOTAL = 1024 * 200
NC, NS, L = 2, 16, 16
NW = NC * NS
ROWS_W = B_TOTAL // NW
GROUP = 128
NGROUP = ROWS_W // GROUP
NBUF = 2
SEQ = 200
NSEG = 3


def _sinusoidal_pe(seq_len, d_model):
    pos = jnp.arange(seq_len, dtype=jnp.float32)[:, None]
    div = jnp.exp(
        jnp.arange(0, d_model, 2, dtype=jnp.float32)
        * (-math.log(10000.0) / d_model)
    )
    pe = jnp.zeros((seq_len, d_model), dtype=jnp.float32)
    pe = pe.at[:, 0::2].set(jnp.sin(pos * div))
    pe = pe.at[:, 1::2].set(jnp.cos(pos * div))
    return pe


def _body(x_hbm, seg_hbm, tok_hbm, comb_hbm, out_hbm,
          idx_v, cidx_v, tok_bufs, comb_bufs, out_bufs, gsems, ssems):
    wid = lax.axis_index("s") * NC + lax.axis_index("c")
    base = wid * ROWS_W

    pltpu.sync_copy(x_hbm.at[pl.ds(base, ROWS_W)], idx_v)
    pltpu.sync_copy(seg_hbm.at[pl.ds(base, ROWS_W)], cidx_v)

    iota = lax.iota(jnp.int32, L)
    def cidx_body(j, carry):
        off = j * L
        posv = (off + iota) % SEQ
        segv = cidx_v[pl.ds(off, L)]
        cidx_v[pl.ds(off, L)] = segv * SEQ + posv
        return carry
    lax.fori_loop(0, ROWS_W // L, cidx_body, 0)

    def gathers_start(g, b):
        pltpu.make_async_copy(
            tok_hbm.at[idx_v.at[pl.ds(g * GROUP, GROUP)]],
            tok_bufs[b], gsems[b]).start()
        pltpu.make_async_copy(
            comb_hbm.at[cidx_v.at[pl.ds(g * GROUP, GROUP)]],
            comb_bufs[b], gsems[b]).start()

    def gathers_wait(g, b):
        pltpu.make_async_copy(
            tok_hbm.at[idx_v.at[pl.ds(g * GROUP, GROUP)]],
            tok_bufs[b], gsems[b]).wait()
        pltpu.make_async_copy(
            comb_hbm.at[cidx_v.at[pl.ds(g * GROUP, GROUP)]],
            comb_bufs[b], gsems[b]).wait()

    def scatter_start(g, b):
        pltpu.make_async_copy(
            out_bufs[b], out_hbm.at[pl.ds(base + g * GROUP, GROUP)],
            ssems[b]).start()

    def scatter_wait(g, b):
        pltpu.make_async_copy(
            out_bufs[b], out_hbm.at[pl.ds(base + g * GROUP, GROUP)],
            ssems[b]).wait()

    for b in range(NBUF):
        gathers_start(b, b)

    def outer(og, carry):
        for b in range(NBUF):
            g = og * NBUF + b
            gathers_wait(g, b)
            @pl.when(og > 0)
            def _():
                scatter_wait(g - NBUF, b)

            def row_body(t, rcarry):
                for c in range(D // L):
                    tv = tok_bufs[b][t, pl.ds(c * L, L)]
                    cv = comb_bufs[b][t, pl.ds(c * L, L)]
                    out_bufs[b][t, pl.ds(c * L, L)] = tv * 8.0 + cv
                return rcarry
            lax.fori_loop(0, GROUP, row_body, 0)

            @pl.when(g + NBUF < NGROUP)
            def _():
                gathers_start(g + NBUF, b)
            scatter_start(g, b)
        return carry
    lax.fori_loop(0, NGROUP // NBUF, outer, 0)

    for b in range(NBUF):
        scatter_wait(NGROUP - NBUF + b, b)


def _sc_embed(x_flat, seg_flat, token_table, comb):
    mesh = plsc.VectorSubcoreMesh(core_axis_name="c", subcore_axis_name="s")

    def body(x_hbm, seg_hbm, tok_hbm, comb_hbm, out_hbm,
             idx_v, cidx_v, tb0, tb1, cb0, cb1, ob0, ob1,
             gs0, gs1, ss0, ss1):
        _body(x_hbm, seg_hbm, tok_hbm, comb_hbm, out_hbm,
              idx_v, cidx_v, (tb0, tb1), (cb0, cb1), (ob0, ob1),
              (gs0, gs1), (ss0, ss1))

    run = pl.kernel(
        body,
        out_type=jax.ShapeDtypeStruct((B_TOTAL, D), jnp.float32),
        mesh=mesh,
        scratch_types=[
            pltpu.VMEM((ROWS_W,), jnp.int32),
            pltpu.VMEM((ROWS_W,), jnp.int32),
            pltpu.VMEM((GROUP, D), jnp.float32),
            pltpu.VMEM((GROUP, D), jnp.float32),
            pltpu.VMEM((GROUP, D), jnp.float32),
            pltpu.VMEM((GROUP, D), jnp.float32),
            pltpu.VMEM((GROUP, D), jnp.float32),
            pltpu.VMEM((GROUP, D), jnp.float32),
            pltpu.SemaphoreType.DMA,
            pltpu.SemaphoreType.DMA,
            pltpu.SemaphoreType.DMA,
            pltpu.SemaphoreType.DMA,
        ],
        compiler_params=pltpu.CompilerParams(
            use_tc_tiling_on_sc=False, needs_layout_passes=False),
    )
    return run(x_flat, seg_flat, token_table, comb)


def kernel(x, segment_label, token_table, segment_table):
    batch, seq = x.shape
    x_flat = x.reshape(-1).astype(jnp.int32)
    seg_flat = segment_label.reshape(-1).astype(jnp.int32)
    pe = _sinusoidal_pe(seq, D)
    comb = (segment_table[:, None, :] + pe[None, :, :]).reshape(NSEG * SEQ, D)
    out = _sc_embed(x_flat, seg_flat, token_table, comb)
    return out.reshape(batch, seq, D)
